# baseline (speedup 1.0000x reference)
import functools

import jax
import jax.numpy as jnp
from jax import lax
from jax.experimental import pallas as pl
from jax.experimental.pallas import tpu as pltpu
from jax.experimental.pallas import tpu_sc as plsc

N = 50000
E = 800000
F = 128
H = 64
HQ = H // 4
G = 32

NC = 2
NS = 16
E_PAD = 819200
EDGES_PER_SUB = E_PAD // NS
CHUNK_E = 400
N_CHUNKS = EDGES_PER_SUB // CHUNK_E
N_PAIRS = N_CHUNKS // 2
N_PAD = 50048
ACC_ROWS = N_PAD
ZROWS = ACC_ROWS // NS
WB_ROWS = ACC_ROWS // NS
MROWS = 3128
MTAIL = N - (NS - 1) * MROWS

BLK = 2000
NBLK = N // BLK


def _sc_scatter_body(src_hbm, dst_hbm, m_hbm, zeros_hbm, out_hbm,
                     srcv0, dstv0, srcv1, dstv1, rows0, rows1, m_s, acc,
                     gsem, ssem):
    c = lax.axis_index("c")
    s = lax.axis_index("s")

    def load_and_gather(j, sv, dv, rv):
        eb = s * EDGES_PER_SUB + j * CHUNK_E
        pltpu.sync_copy(src_hbm.at[pl.ds(eb, CHUNK_E)], sv)
        pltpu.sync_copy(dst_hbm.at[pl.ds(eb, CHUNK_E)], dv)
        pltpu.async_copy(m_s.at[sv], rv, gsem)

    def drain(sem, rv):
        pltpu.make_async_copy(m_s.at[pl.ds(0, CHUNK_E)], rv, sem).wait()

    for p in range(2):
        q = 2 * c + p

        pltpu.sync_copy(zeros_hbm.at[pl.ds(0, ZROWS)],
                        acc.at[pl.ds(s * ZROWS, ZROWS)])

        @pl.when(s < NS - 1)
        def _():
            pltpu.sync_copy(m_hbm.at[pl.ds(q * N + s * MROWS, MROWS)],
                            m_s.at[pl.ds(s * MROWS, MROWS)])

        @pl.when(s == NS - 1)
        def _():
            pltpu.sync_copy(m_hbm.at[pl.ds(q * N + (NS - 1) * MROWS, MTAIL)],
                            m_s.at[pl.ds((NS - 1) * MROWS, MTAIL)])

        plsc.subcore_barrier()

        load_and_gather(0, srcv0, dstv0, rows0)

        def pair_body(jj, carry):
            j = 2 * jj

            @pl.when(jj > 0)
            def _():
                drain(ssem, rows1)
            load_and_gather(j + 1, srcv1, dstv1, rows1)
            drain(gsem, rows0)
            pltpu.async_copy(rows0, acc.at[dstv0], ssem, add=True)

            @pl.when(jj < N_PAIRS - 1)
            def _():
                drain(ssem, rows0)
                load_and_gather(j + 2, srcv0, dstv0, rows0)
            drain(gsem, rows1)
            pltpu.async_copy(rows1, acc.at[dstv1], ssem, add=True)
            return carry

        lax.fori_loop(0, N_PAIRS, pair_body, 0)
        drain(ssem, rows0)
        drain(ssem, rows1)
        plsc.subcore_barrier()

        pltpu.sync_copy(acc.at[pl.ds(s * WB_ROWS, WB_ROWS)],
                        out_hbm.at[pl.ds(q * N_PAD + s * WB_ROWS, WB_ROWS)])


@functools.cache
def _get_scatter_call():
    mesh = plsc.VectorSubcoreMesh(core_axis_name="c", subcore_axis_name="s",
                                  num_cores=NC, num_subcores=NS)
    return pl.kernel(
        _sc_scatter_body,
        out_type=jax.ShapeDtypeStruct((4 * N_PAD, HQ), jnp.float32),
        mesh=mesh,
        compiler_params=pltpu.CompilerParams(use_tc_tiling_on_sc=False),
        scratch_types=[
            pltpu.VMEM((CHUNK_E,), jnp.int32),
            pltpu.VMEM((CHUNK_E,), jnp.int32),
            pltpu.VMEM((CHUNK_E,), jnp.int32),
            pltpu.VMEM((CHUNK_E,), jnp.int32),
            pltpu.VMEM((CHUNK_E, HQ), jnp.float32),
            pltpu.VMEM((CHUNK_E, HQ), jnp.float32),
            pltpu.VMEM_SHARED((N, HQ), jnp.float32),
            pltpu.VMEM_SHARED((ACC_ROWS, HQ), jnp.float32),
            pltpu.SemaphoreType.DMA,
            pltpu.SemaphoreType.DMA,
        ],
    )


def _embed_body(x_ref, win_ref, w0_ref, x0_ref, m_ref):
    xe = jnp.tanh(lax.dot_general(x_ref[...], win_ref[...],
                                  (((1,), (1,)), ((), ())),
                                  preferred_element_type=jnp.float32))
    x0_ref[...] = xe
    w0 = w0_ref[...]
    for q in range(4):
        m_ref[q, :, :] = jnp.dot(xe, w0[:, q * HQ:(q + 1) * HQ],
                                 preferred_element_type=jnp.float32)


_embed_call = pl.pallas_call(
    _embed_body,
    grid=(NBLK,),
    in_specs=[
        pl.BlockSpec((BLK, F), lambda i: (i, 0)),
        pl.BlockSpec((H, F), lambda i: (0, 0)),
        pl.BlockSpec((H, H), lambda i: (0, 0)),
    ],
    out_specs=[
        pl.BlockSpec((BLK, H), lambda i: (i, 0)),
        pl.BlockSpec((4, BLK, HQ), lambda i: (0, i, 0)),
    ],
    out_shape=[
        jax.ShapeDtypeStruct((N, H), jnp.float32),
        jax.ShapeDtypeStruct((4, N, HQ), jnp.float32),
    ],
)


def _gru_math(agg0, agg1, agg2, agg3, h, wih, whh, bih, bhh):
    agg = jnp.concatenate([agg0, agg1, agg2, agg3], axis=1)
    gi = lax.dot_general(agg, wih, (((1,), (1,)), ((), ())),
                         preferred_element_type=jnp.float32) + bih
    gh = lax.dot_general(h, whh, (((1,), (1,)), ((), ())),
                         preferred_element_type=jnp.float32) + bhh
    r = jax.nn.sigmoid(gi[:, :H] + gh[:, :H])
    z = jax.nn.sigmoid(gi[:, H:2 * H] + gh[:, H:2 * H])
    n = jnp.tanh(gi[:, 2 * H:] + r * gh[:, 2 * H:])
    return (1.0 - z) * n + z * h


def _gru_body(agg0_ref, agg1_ref, agg2_ref, agg3_ref, x_ref, wih_ref,
              whh_ref, bih_ref, bhh_ref, wn_ref, x1_ref, m_ref):
    x1 = _gru_math(agg0_ref[...], agg1_ref[...], agg2_ref[...], agg3_ref[...],
                   x_ref[...], wih_ref[...], whh_ref[...], bih_ref[...],
                   bhh_ref[...])
    x1_ref[...] = x1
    wn = wn_ref[...]
    for q in range(4):
        m_ref[q, :, :] = jnp.dot(x1, wn[:, q * HQ:(q + 1) * HQ],
                                 preferred_element_type=jnp.float32)


def _gru_last_body(agg0_ref, agg1_ref, agg2_ref, agg3_ref, x_ref, wih_ref,
                   whh_ref, bih_ref, bhh_ref, x1_ref):
    x1_ref[...] = _gru_math(agg0_ref[...], agg1_ref[...], agg2_ref[...],
                            agg3_ref[...], x_ref[...], wih_ref[...],
                            whh_ref[...], bih_ref[...], bhh_ref[...])


_GRU_IN_SPECS = [
    pl.BlockSpec((BLK, HQ), lambda i: (i, 0)),
    pl.BlockSpec((BLK, HQ), lambda i: (i, 0)),
    pl.BlockSpec((BLK, HQ), lambda i: (i, 0)),
    pl.BlockSpec((BLK, HQ), lambda i: (i, 0)),
    pl.BlockSpec((BLK, H), lambda i: (i, 0)),
    pl.BlockSpec((3 * H, H), lambda i: (0, 0)),
    pl.BlockSpec((3 * H, H), lambda i: (0, 0)),
    pl.BlockSpec((1, 3 * H), lambda i: (0, 0)),
    pl.BlockSpec((1, 3 * H), lambda i: (0, 0)),
]

_gru_call = pl.pallas_call(
    _gru_body,
    grid=(NBLK,),
    in_specs=_GRU_IN_SPECS + [pl.BlockSpec((H, H), lambda i: (0, 0))],
    out_specs=[
        pl.BlockSpec((BLK, H), lambda i: (i, 0)),
        pl.BlockSpec((4, BLK, HQ), lambda i: (0, i, 0)),
    ],
    out_shape=[
        jax.ShapeDtypeStruct((N, H), jnp.float32),
        jax.ShapeDtypeStruct((4, N, HQ), jnp.float32),
    ],
)

_gru_last_call = pl.pallas_call(
    _gru_last_body,
    grid=(NBLK,),
    in_specs=_GRU_IN_SPECS,
    out_specs=pl.BlockSpec((BLK, H), lambda i: (i, 0)),
    out_shape=jax.ShapeDtypeStruct((N, H), jnp.float32),
)


def _pool_body(ax_ref, vx_ref, batch_ref, pw_ref, pb_ref, out_ref,
               sums_ref, cnt_ref):
    i = pl.program_id(0)

    @pl.when(i == 0)
    def _():
        sums_ref[...] = jnp.zeros((G, F), jnp.float32)
        cnt_ref[...] = jnp.zeros((G, F), jnp.float32)

    b = batch_ref[0, 0, :]
    g_iota = lax.broadcasted_iota(jnp.int32, (G, BLK), 0)
    mask = (jnp.broadcast_to(b[None, :], (G, BLK)) == g_iota).astype(jnp.float32)
    both = jnp.concatenate([ax_ref[...], vx_ref[...]], axis=1)
    sums_ref[...] += jnp.dot(mask, both, preferred_element_type=jnp.float32)
    cnt_ref[...] += jnp.broadcast_to(
        jnp.sum(mask, axis=1, keepdims=True), (G, F))

    @pl.when(i == NBLK - 1)
    def _():
        mean = sums_ref[...] / jnp.maximum(cnt_ref[...], 1.0)
        h = jnp.maximum(mean, 0.0)
        o = lax.dot_general(h, pw_ref[...], (((1,), (1,)), ((), ())),
                            preferred_element_type=jnp.float32)
        out_ref[...] = jnp.broadcast_to(o, (G, F)) + pb_ref[0, 0]


_pool_call = pl.pallas_call(
    _pool_body,
    grid=(NBLK,),
    in_specs=[
        pl.BlockSpec((BLK, H), lambda i: (i, 0)),
        pl.BlockSpec((BLK, H), lambda i: (i, 0)),
        pl.BlockSpec((1, 1, BLK), lambda i: (i, 0, 0)),
        pl.BlockSpec((1, F), lambda i: (0, 0)),
        pl.BlockSpec((1, 1), lambda i: (0, 0)),
    ],
    out_specs=pl.BlockSpec((G, F), lambda i: (0, 0)),
    out_shape=jax.ShapeDtypeStruct((G, F), jnp.float32),
    scratch_shapes=[
        pltpu.VMEM((G, F), jnp.float32),
        pltpu.VMEM((G, F), jnp.float32),
    ],
)


def _prep_edges(edge_index):
    src = edge_index[0]
    dst = edge_index[1]
    pad = E_PAD - E
    src_p = jnp.concatenate([src, jnp.zeros((pad,), jnp.int32)])
    dst_p = jnp.concatenate([dst, jnp.full((pad,), N, jnp.int32)])
    return src_p, dst_p


def kernel(atom_x, voro_x, atom_edge_index, voro_edge_index, batch,
           atom_in_W, voro_in_W,
           atom_weight, atom_W_ih, atom_W_hh, atom_b_ih, atom_b_hh,
           voro_weight, voro_W_ih, voro_W_hh, voro_b_ih, voro_b_hh,
           pred_W, pred_b):
    zeros = jnp.zeros((ZROWS, HQ), jnp.float32)
    scatter = _get_scatter_call()
    src_a, dst_a = _prep_edges(atom_edge_index)
    src_v, dst_v = _prep_edges(voro_edge_index)
    abih = atom_b_ih.reshape(1, 3 * H)
    abhh = atom_b_hh.reshape(1, 3 * H)
    vbih = voro_b_ih.reshape(1, 3 * H)
    vbhh = voro_b_hh.reshape(1, 3 * H)

    xa, ma = _embed_call(atom_x, atom_in_W, atom_weight[0])
    xv, mv = _embed_call(voro_x, voro_in_W, voro_weight[0])
    for l in range(3):
        agg_a = scatter(src_a, dst_a, ma.reshape(4 * N, HQ), zeros)
        agg_v = scatter(src_v, dst_v, mv.reshape(4 * N, HQ), zeros)
        aq = [agg_a[q * N_PAD:q * N_PAD + N] for q in range(4)]
        vq = [agg_v[q * N_PAD:q * N_PAD + N] for q in range(4)]
        if l < 2:
            xa, ma = _gru_call(aq[0], aq[1], aq[2], aq[3], xa, atom_W_ih,
                               atom_W_hh, abih, abhh, atom_weight[l + 1])
            xv, mv = _gru_call(vq[0], vq[1], vq[2], vq[3], xv, voro_W_ih,
                               voro_W_hh, vbih, vbhh, voro_weight[l + 1])
        else:
            xa = _gru_last_call(aq[0], aq[1], aq[2], aq[3], xa, atom_W_ih,
                                atom_W_hh, abih, abhh)
            xv = _gru_last_call(vq[0], vq[1], vq[2], vq[3], xv, voro_W_ih,
                                voro_W_hh, vbih, vbhh)
    batch3 = batch.reshape(NBLK, 1, BLK)
    out = _pool_call(xa, xv, batch3, pred_W.reshape(1, F),
                     pred_b.reshape(1, 1))
    return out[:, 0]


# speedup vs baseline: 1.9827x; 1.9827x over previous
import functools

import jax
import jax.numpy as jnp
from jax import lax
from jax.experimental import pallas as pl
from jax.experimental.pallas import tpu as pltpu
from jax.experimental.pallas import tpu_sc as plsc

N = 50000
E = 800000
F = 128
H = 64
HH = H // 2
G = 32

NC = 2
NS = 16
E_PAD = 819200
EDGES_PER_SUB = E_PAD // NS
CHUNK_E = 400
N_CHUNKS = EDGES_PER_SUB // CHUNK_E
N_PAIRS = N_CHUNKS // 2
N_PAD = 50048
ACC_ROWS = N_PAD
ZROWS = ACC_ROWS // NS
WB_ROWS = ACC_ROWS // NS
MROWS = 3128
MTAIL = N - (NS - 1) * MROWS

BLK = 2000
NBLK = N // BLK


def _sc_scatter_body(src_hbm, dst_hbm, m_hbm, zeros_hbm, out_hbm,
                     srcv0, dstv0, srcv1, dstv1, rows0, rows1, m_s, acc,
                     gsem, ssem):
    c = lax.axis_index("c")
    s = lax.axis_index("s")

    def load_and_gather(j, sv, dv, rv):
        eb = s * EDGES_PER_SUB + j * CHUNK_E
        pltpu.sync_copy(src_hbm.at[pl.ds(eb, CHUNK_E)], sv)
        pltpu.sync_copy(dst_hbm.at[pl.ds(eb, CHUNK_E)], dv)
        pltpu.async_copy(m_s.at[sv], rv, gsem)

    def drain(sem, rv):
        pltpu.make_async_copy(m_s.at[pl.ds(0, CHUNK_E)], rv, sem).wait()

    pltpu.sync_copy(zeros_hbm.at[pl.ds(0, ZROWS)],
                    acc.at[pl.ds(s * ZROWS, ZROWS)])

    @pl.when(s < NS - 1)
    def _():
        pltpu.sync_copy(m_hbm.at[pl.ds(c * N + s * MROWS, MROWS)],
                        m_s.at[pl.ds(s * MROWS, MROWS)])

    @pl.when(s == NS - 1)
    def _():
        pltpu.sync_copy(m_hbm.at[pl.ds(c * N + (NS - 1) * MROWS, MTAIL)],
                        m_s.at[pl.ds((NS - 1) * MROWS, MTAIL)])

    plsc.subcore_barrier()

    load_and_gather(0, srcv0, dstv0, rows0)

    def pair_body(jj, carry):
        j = 2 * jj

        @pl.when(jj > 0)
        def _():
            drain(ssem, rows1)
        load_and_gather(j + 1, srcv1, dstv1, rows1)
        drain(gsem, rows0)
        pltpu.async_copy(rows0, acc.at[dstv0], ssem, add=True)

        @pl.when(jj < N_PAIRS - 1)
        def _():
            drain(ssem, rows0)
            load_and_gather(j + 2, srcv0, dstv0, rows0)
        drain(gsem, rows1)
        pltpu.async_copy(rows1, acc.at[dstv1], ssem, add=True)
        return carry

    lax.fori_loop(0, N_PAIRS, pair_body, 0)
    drain(ssem, rows0)
    drain(ssem, rows1)
    plsc.subcore_barrier()

    pltpu.sync_copy(acc.at[pl.ds(s * WB_ROWS, WB_ROWS)],
                    out_hbm.at[pl.ds(c * N_PAD + s * WB_ROWS, WB_ROWS)])


@functools.cache
def _get_scatter_call():
    mesh = plsc.VectorSubcoreMesh(core_axis_name="c", subcore_axis_name="s",
                                  num_cores=NC, num_subcores=NS)
    return pl.kernel(
        _sc_scatter_body,
        out_type=jax.ShapeDtypeStruct((2 * N_PAD, HH), jnp.bfloat16),
        mesh=mesh,
        compiler_params=pltpu.CompilerParams(use_tc_tiling_on_sc=False),
        scratch_types=[
            pltpu.VMEM((CHUNK_E,), jnp.int32),
            pltpu.VMEM((CHUNK_E,), jnp.int32),
            pltpu.VMEM((CHUNK_E,), jnp.int32),
            pltpu.VMEM((CHUNK_E,), jnp.int32),
            pltpu.VMEM((CHUNK_E, HH), jnp.bfloat16),
            pltpu.VMEM((CHUNK_E, HH), jnp.bfloat16),
            pltpu.VMEM_SHARED((N, HH), jnp.bfloat16),
            pltpu.VMEM_SHARED((ACC_ROWS, HH), jnp.bfloat16),
            pltpu.SemaphoreType.DMA,
            pltpu.SemaphoreType.DMA,
        ],
    )


def _embed_body(x_ref, win_ref, w0_ref, x0_ref, m_ref):
    xe = jnp.tanh(lax.dot_general(x_ref[...], win_ref[...],
                                  (((1,), (1,)), ((), ())),
                                  preferred_element_type=jnp.float32))
    x0_ref[...] = xe
    w0 = w0_ref[...]
    for h in range(2):
        m_ref[h, :, :] = jnp.dot(
            xe, w0[:, h * HH:(h + 1) * HH],
            preferred_element_type=jnp.float32).astype(jnp.bfloat16)


_embed_call = pl.pallas_call(
    _embed_body,
    grid=(NBLK,),
    in_specs=[
        pl.BlockSpec((BLK, F), lambda i: (i, 0)),
        pl.BlockSpec((H, F), lambda i: (0, 0)),
        pl.BlockSpec((H, H), lambda i: (0, 0)),
    ],
    out_specs=[
        pl.BlockSpec((BLK, H), lambda i: (i, 0)),
        pl.BlockSpec((2, BLK, HH), lambda i: (0, i, 0)),
    ],
    out_shape=[
        jax.ShapeDtypeStruct((N, H), jnp.float32),
        jax.ShapeDtypeStruct((2, N, HH), jnp.bfloat16),
    ],
)


def _gru_math(agg0, agg1, h, wih, whh, bih, bhh):
    agg = jnp.concatenate([agg0, agg1], axis=1).astype(jnp.float32)
    gi = lax.dot_general(agg, wih, (((1,), (1,)), ((), ())),
                         preferred_element_type=jnp.float32) + bih
    gh = lax.dot_general(h, whh, (((1,), (1,)), ((), ())),
                         preferred_element_type=jnp.float32) + bhh
    r = jax.nn.sigmoid(gi[:, :H] + gh[:, :H])
    z = jax.nn.sigmoid(gi[:, H:2 * H] + gh[:, H:2 * H])
    n = jnp.tanh(gi[:, 2 * H:] + r * gh[:, 2 * H:])
    return (1.0 - z) * n + z * h


def _gru_body(agg0_ref, agg1_ref, x_ref, wih_ref, whh_ref, bih_ref, bhh_ref,
              wn_ref, x1_ref, m_ref):
    x1 = _gru_math(agg0_ref[...], agg1_ref[...], x_ref[...], wih_ref[...],
                   whh_ref[...], bih_ref[...], bhh_ref[...])
    x1_ref[...] = x1
    wn = wn_ref[...]
    for h in range(2):
        m_ref[h, :, :] = jnp.dot(
            x1, wn[:, h * HH:(h + 1) * HH],
            preferred_element_type=jnp.float32).astype(jnp.bfloat16)


def _gru_last_body(agg0_ref, agg1_ref, x_ref, wih_ref, whh_ref, bih_ref,
                   bhh_ref, x1_ref):
    x1_ref[...] = _gru_math(agg0_ref[...], agg1_ref[...], x_ref[...],
                            wih_ref[...], whh_ref[...], bih_ref[...],
                            bhh_ref[...])


_GRU_IN_SPECS = [
    pl.BlockSpec((BLK, HH), lambda i: (i, 0)),
    pl.BlockSpec((BLK, HH), lambda i: (i, 0)),
    pl.BlockSpec((BLK, H), lambda i: (i, 0)),
    pl.BlockSpec((3 * H, H), lambda i: (0, 0)),
    pl.BlockSpec((3 * H, H), lambda i: (0, 0)),
    pl.BlockSpec((1, 3 * H), lambda i: (0, 0)),
    pl.BlockSpec((1, 3 * H), lambda i: (0, 0)),
]

_gru_call = pl.pallas_call(
    _gru_body,
    grid=(NBLK,),
    in_specs=_GRU_IN_SPECS + [pl.BlockSpec((H, H), lambda i: (0, 0))],
    out_specs=[
        pl.BlockSpec((BLK, H), lambda i: (i, 0)),
        pl.BlockSpec((2, BLK, HH), lambda i: (0, i, 0)),
    ],
    out_shape=[
        jax.ShapeDtypeStruct((N, H), jnp.float32),
        jax.ShapeDtypeStruct((2, N, HH), jnp.bfloat16),
    ],
)

_gru_last_call = pl.pallas_call(
    _gru_last_body,
    grid=(NBLK,),
    in_specs=_GRU_IN_SPECS,
    out_specs=pl.BlockSpec((BLK, H), lambda i: (i, 0)),
    out_shape=jax.ShapeDtypeStruct((N, H), jnp.float32),
)


def _pool_body(ax_ref, vx_ref, batch_ref, pw_ref, pb_ref, out_ref,
               sums_ref, cnt_ref):
    i = pl.program_id(0)

    @pl.when(i == 0)
    def _():
        sums_ref[...] = jnp.zeros((G, F), jnp.float32)
        cnt_ref[...] = jnp.zeros((G, F), jnp.float32)

    b = batch_ref[0, 0, :]
    g_iota = lax.broadcasted_iota(jnp.int32, (G, BLK), 0)
    mask = (jnp.broadcast_to(b[None, :], (G, BLK)) == g_iota).astype(jnp.float32)
    both = jnp.concatenate([ax_ref[...], vx_ref[...]], axis=1)
    sums_ref[...] += jnp.dot(mask, both, preferred_element_type=jnp.float32)
    cnt_ref[...] += jnp.broadcast_to(
        jnp.sum(mask, axis=1, keepdims=True), (G, F))

    @pl.when(i == NBLK - 1)
    def _():
        mean = sums_ref[...] / jnp.maximum(cnt_ref[...], 1.0)
        h = jnp.maximum(mean, 0.0)
        o = lax.dot_general(h, pw_ref[...], (((1,), (1,)), ((), ())),
                            preferred_element_type=jnp.float32)
        out_ref[...] = jnp.broadcast_to(o, (G, F)) + pb_ref[0, 0]


_pool_call = pl.pallas_call(
    _pool_body,
    grid=(NBLK,),
    in_specs=[
        pl.BlockSpec((BLK, H), lambda i: (i, 0)),
        pl.BlockSpec((BLK, H), lambda i: (i, 0)),
        pl.BlockSpec((1, 1, BLK), lambda i: (i, 0, 0)),
        pl.BlockSpec((1, F), lambda i: (0, 0)),
        pl.BlockSpec((1, 1), lambda i: (0, 0)),
    ],
    out_specs=pl.BlockSpec((G, F), lambda i: (0, 0)),
    out_shape=jax.ShapeDtypeStruct((G, F), jnp.float32),
    scratch_shapes=[
        pltpu.VMEM((G, F), jnp.float32),
        pltpu.VMEM((G, F), jnp.float32),
    ],
)


def _prep_edges(edge_index):
    src = edge_index[0]
    dst = edge_index[1]
    pad = E_PAD - E
    src_p = jnp.concatenate([src, jnp.zeros((pad,), jnp.int32)])
    dst_p = jnp.concatenate([dst, jnp.full((pad,), N, jnp.int32)])
    return src_p, dst_p


def kernel(atom_x, voro_x, atom_edge_index, voro_edge_index, batch,
           atom_in_W, voro_in_W,
           atom_weight, atom_W_ih, atom_W_hh, atom_b_ih, atom_b_hh,
           voro_weight, voro_W_ih, voro_W_hh, voro_b_ih, voro_b_hh,
           pred_W, pred_b):
    zeros = jnp.zeros((ZROWS, HH), jnp.bfloat16)
    scatter = _get_scatter_call()
    src_a, dst_a = _prep_edges(atom_edge_index)
    src_v, dst_v = _prep_edges(voro_edge_index)
    abih = atom_b_ih.reshape(1, 3 * H)
    abhh = atom_b_hh.reshape(1, 3 * H)
    vbih = voro_b_ih.reshape(1, 3 * H)
    vbhh = voro_b_hh.reshape(1, 3 * H)

    xa, ma = _embed_call(atom_x, atom_in_W, atom_weight[0])
    xv, mv = _embed_call(voro_x, voro_in_W, voro_weight[0])
    for l in range(3):
        agg_a = scatter(src_a, dst_a, ma.reshape(2 * N, HH), zeros)
        agg_v = scatter(src_v, dst_v, mv.reshape(2 * N, HH), zeros)
        aq = [agg_a[q * N_PAD:q * N_PAD + N] for q in range(2)]
        vq = [agg_v[q * N_PAD:q * N_PAD + N] for q in range(2)]
        if l < 2:
            xa, ma = _gru_call(aq[0], aq[1], xa, atom_W_ih,
                               atom_W_hh, abih, abhh, atom_weight[l + 1])
            xv, mv = _gru_call(vq[0], vq[1], xv, voro_W_ih,
                               voro_W_hh, vbih, vbhh, voro_weight[l + 1])
        else:
            xa = _gru_last_call(aq[0], aq[1], xa, atom_W_ih,
                                atom_W_hh, abih, abhh)
            xv = _gru_last_call(vq[0], vq[1], xv, voro_W_ih,
                                voro_W_hh, vbih, vbhh)
    batch3 = batch.reshape(NBLK, 1, BLK)
    out = _pool_call(xa, xv, batch3, pred_W.reshape(1, F),
                     pred_b.reshape(1, 1))
    return out[:, 0]
